# baseline (speedup 1.0000x reference)
import jax
import jax.numpy as jnp
from jax import lax
from jax.experimental import pallas as pl
from jax.experimental.pallas import tpu as pltpu
from jax.experimental.pallas import tpu_sc as plsc

T = 2048
D = 1024
H = 2048
E = 8
TILE_M = 256
NROWS = T * 2 + E * TILE_M
NTILES = NROWS // TILE_M
CSBLK = 256

NC = 2
NS = 16
NW = NC * NS
TPW = T // NW
CHUNK = 32


def _router_body(x_ref, wg_ref, dp_ref, gp_ref, ef_ref, sw_ref, nxt_ref,
                 slot_ref):
    x = x_ref[...]
    wg = wg_ref[...]
    logits = jnp.dot(x, wg, preferred_element_type=jnp.float32)
    m = jnp.max(logits, axis=1, keepdims=True)
    ex = jnp.exp(logits - m)
    probs = ex / jnp.sum(ex, axis=1, keepdims=True)

    eids = lax.broadcasted_iota(jnp.int32, (T, E), 1)
    v0 = jnp.max(probs, axis=1, keepdims=True)
    i0 = jnp.min(jnp.where(probs == v0, eids, E), axis=1, keepdims=True)
    oh0 = (eids == i0).astype(jnp.float32)
    probs1 = jnp.where(eids == i0, -1.0, probs)
    v1 = jnp.max(probs1, axis=1, keepdims=True)
    i1 = jnp.min(jnp.where(probs1 == v1, eids, E), axis=1, keepdims=True)
    oh1 = (eids == i1).astype(jnp.float32)

    s = v0 + v1

    mem = oh0 + oh1
    li = lax.broadcasted_iota(jnp.int32, (CSBLK, CSBLK), 0)
    lj = lax.broadcasted_iota(jnp.int32, (CSBLK, CSBLK), 1)
    ltri = (li >= lj).astype(jnp.float32)
    carry = jnp.zeros((1, E), dtype=jnp.float32)
    blocks = []
    for b in range(T // CSBLK):
        mb = lax.slice(mem, (b * CSBLK, 0), ((b + 1) * CSBLK, E))
        cb = jnp.dot(ltri, mb, preferred_element_type=jnp.float32) + carry
        carry = lax.slice(cb, (CSBLK - 1, 0), (CSBLK, E))
        blocks.append(cb)
    csum = jnp.concatenate(blocks, axis=0)

    counts = carry
    padded = (jnp.floor((counts + (TILE_M - 1)) * (1.0 / TILE_M))) * TILE_M
    ei = lax.broadcasted_iota(jnp.int32, (E, E), 0)
    ej = lax.broadcasted_iota(jnp.int32, (E, E), 1)
    utri = (ei <= ej).astype(jnp.float32)
    ends = jnp.dot(padded, utri, preferred_element_type=jnp.float32)
    offsets = ends - padded

    off0 = jnp.sum(offsets * oh0, axis=1, keepdims=True)
    off1 = jnp.sum(offsets * oh1, axis=1, keepdims=True)
    c0 = jnp.sum(csum * oh0, axis=1, keepdims=True)
    c1 = jnp.sum(csum * oh1, axis=1, keepdims=True)
    d0 = (off0 + c0).astype(jnp.int32) - 1
    d1 = (off1 + c1).astype(jnp.int32) - 1
    dp_ref[...] = jnp.concatenate(
        [d0.reshape(NW, TPW), d1.reshape(NW, TPW)], axis=1)
    gp_ref[...] = jnp.concatenate(
        [(v0 / s).reshape(NW, TPW), (v1 / s).reshape(NW, TPW)], axis=1)

    tstart = (lax.broadcasted_iota(jnp.int32, (NTILES, E), 0)
              * TILE_M).astype(jnp.float32)
    ef = jnp.minimum(
        jnp.sum((tstart >= ends).astype(jnp.int32), axis=1, keepdims=True),
        E - 1)
    ef_ref[...] = ef

    efprev = jnp.concatenate(
        [jnp.full((1, 1), -1, jnp.int32),
         lax.slice(ef, (0, 0), (NTILES - 1, 1))], axis=0)
    sw_ref[...] = (ef != efprev).astype(jnp.int32)

    ohT = (lax.broadcasted_iota(jnp.int32, (NTILES, E), 1)
           == ef).astype(jnp.float32)
    present = jnp.where(
        (padded > 0.5)
        | (lax.broadcasted_iota(jnp.int32, (1, E), 1) == E - 1), 1.0, 0.0)
    exc = jnp.dot(present, (ei < ej).astype(jnp.float32),
                  preferred_element_type=jnp.float32)
    rid = jnp.sum(exc * ohT, axis=1, keepdims=True).astype(jnp.int32)
    slot_ref[...] = lax.rem(rid, 2)

    eye = (ei == ej).astype(jnp.float32)
    present_col = lax.dot_general(eye, present,
                                  (((1,), (1,)), ((), ())))
    valt = jnp.where((ei > ej) & (present_col > 0.5),
                     ei.astype(jnp.float32), float(E))
    nxt_of = jnp.min(valt, axis=0, keepdims=True)
    nxt_of = jnp.where(nxt_of > E - 0.5, -1.0, nxt_of)
    nxt_ref[...] = jnp.sum(nxt_of * ohT, axis=1, keepdims=True).astype(
        jnp.int32)


def _router(x, wg):
    return pl.pallas_call(
        _router_body,
        out_shape=[
            jax.ShapeDtypeStruct((NW, 2 * TPW), jnp.int32),
            jax.ShapeDtypeStruct((NW, 2 * TPW), jnp.float32),
            jax.ShapeDtypeStruct((NTILES, 1), jnp.int32),
            jax.ShapeDtypeStruct((NTILES, 1), jnp.int32),
            jax.ShapeDtypeStruct((NTILES, 1), jnp.int32),
            jax.ShapeDtypeStruct((NTILES, 1), jnp.int32),
        ],
        compiler_params=pltpu.CompilerParams(
            vmem_limit_bytes=60 * 1024 * 1024),
    )(x, wg)


def _dispatch_body(x_hbm, dp_hbm, gp_hbm, xin_hbm, garr_hbm,
                   xv, ipk, dpv, gpv, garr_v):
    cid = lax.axis_index("core")
    sid = lax.axis_index("subcore")
    wid = sid * NC + cid
    base = pl.multiple_of(wid * TPW, TPW)

    pltpu.sync_copy(x_hbm.at[pl.ds(base, TPW)], xv)
    pltpu.sync_copy(dp_hbm.at[pl.ds(wid, 1)], ipk)
    for j in range(TPW // 16):
        rows = xv.at[pl.ds(16 * j, 16)]
        pltpu.sync_copy(rows, xin_hbm.at[ipk[0, pl.ds(16 * j, 16)]])
        pltpu.sync_copy(rows, xin_hbm.at[ipk[0, pl.ds(TPW + 16 * j, 16)]])

    @pl.when(wid == 0)
    def _():
        pltpu.sync_copy(dp_hbm, dpv)
        pltpu.sync_copy(gp_hbm, gpv)
        for w in range(NW):
            for j in range(2 * TPW // 16):
                sl = pl.ds(16 * j, 16)
                plsc.store_scatter(garr_v, [dpv[w, sl]], gpv[w, sl])
        pltpu.sync_copy(garr_v, garr_hbm)


def _dispatch(x, dpack, gpack):
    mesh = plsc.VectorSubcoreMesh(core_axis_name="core",
                                  subcore_axis_name="subcore")
    f = pl.kernel(
        _dispatch_body,
        out_type=[
            jax.ShapeDtypeStruct((NROWS, D), jnp.float32),
            jax.ShapeDtypeStruct((NROWS,), jnp.float32),
        ],
        mesh=mesh,
        scratch_types=[
            pltpu.VMEM((TPW, D), jnp.float32),
            pltpu.VMEM((1, 2 * TPW), jnp.int32),
            pltpu.VMEM((NW, 2 * TPW), jnp.int32),
            pltpu.VMEM((NW, 2 * TPW), jnp.float32),
            pltpu.VMEM((NROWS,), jnp.float32),
        ],
        compiler_params=pltpu.CompilerParams(needs_layout_passes=False),
    )
    return f(x, dpack, gpack)


def _ffn_body(ef_ref, sw_ref, nxt_ref, slot_ref,
              x_ref, g_ref, w1_hbm, b1_ref, w2_hbm, b2_ref, o_ref,
              w1v, w2v, sem1, sem2):
    i = pl.program_id(0)
    e = ef_ref[i]
    sl = slot_ref[i]

    @pl.when(i == 0)
    def _():
        pltpu.make_async_copy(w1_hbm.at[e], w1v.at[sl], sem1.at[sl]).start()
        pltpu.make_async_copy(w2_hbm.at[e], w2v.at[sl], sem2.at[sl]).start()

    @pl.when(sw_ref[i] == 1)
    def _():
        pltpu.make_async_copy(w1_hbm.at[e], w1v.at[sl], sem1.at[sl]).wait()
        pltpu.make_async_copy(w2_hbm.at[e], w2v.at[sl], sem2.at[sl]).wait()
        ne = nxt_ref[i]

        @pl.when(ne >= 0)
        def _():
            osl = 1 - sl
            pltpu.make_async_copy(w1_hbm.at[ne], w1v.at[osl],
                                  sem1.at[osl]).start()
            pltpu.make_async_copy(w2_hbm.at[ne], w2v.at[osl],
                                  sem2.at[osl]).start()

    h = jnp.dot(x_ref[...], w1v[sl], preferred_element_type=jnp.float32)
    h = jax.nn.gelu(h + b1_ref[e])
    y = jnp.dot(h, w2v[sl], preferred_element_type=jnp.float32)
    o_ref[...] = (y + b2_ref[e]) * g_ref[...]


def _ffn(ef, sw, nxt, slot, xin, garr, w1, b1, w2, b2):
    grid_spec = pltpu.PrefetchScalarGridSpec(
        num_scalar_prefetch=4,
        grid=(NTILES,),
        in_specs=[
            pl.BlockSpec((TILE_M, D), lambda i, *_: (i, 0)),
            pl.BlockSpec((TILE_M, 1), lambda i, *_: (i, 0)),
            pl.BlockSpec(memory_space=pl.ANY),
            pl.BlockSpec((E, 1, H), lambda i, *_: (0, 0, 0)),
            pl.BlockSpec(memory_space=pl.ANY),
            pl.BlockSpec((E, 1, D), lambda i, *_: (0, 0, 0)),
        ],
        out_specs=pl.BlockSpec((TILE_M, D), lambda i, *_: (i, 0)),
        scratch_shapes=[
            pltpu.VMEM((2, D, H), jnp.float32),
            pltpu.VMEM((2, H, D), jnp.float32),
            pltpu.SemaphoreType.DMA((2,)),
            pltpu.SemaphoreType.DMA((2,)),
        ],
    )
    return pl.pallas_call(
        _ffn_body,
        grid_spec=grid_spec,
        out_shape=jax.ShapeDtypeStruct((NROWS, D), jnp.float32),
        compiler_params=pltpu.CompilerParams(
            dimension_semantics=("arbitrary",),
            vmem_limit_bytes=60 * 1024 * 1024),
    )(ef, sw, nxt, slot, xin, garr, w1, b1, w2, b2)


def _combine_body(yg_hbm, dp_hbm, out_hbm, ipk, ya, yb):
    cid = lax.axis_index("core")
    sid = lax.axis_index("subcore")
    wid = sid * NC + cid
    base = pl.multiple_of(wid * TPW, TPW)

    pltpu.sync_copy(dp_hbm.at[pl.ds(wid, 1)], ipk)
    for c in range(TPW // CHUNK):
        ia = ipk.at[0, pl.ds(c * CHUNK, CHUNK)]
        ib = ipk.at[0, pl.ds(TPW + c * CHUNK, CHUNK)]
        pltpu.sync_copy(yg_hbm.at[ia], ya)
        pltpu.sync_copy(yg_hbm.at[ib], yb)

        @pl.loop(0, CHUNK)
        def _(r):
            for cc in range(0, D, 16):
                sl = pl.ds(cc, 16)
                ya.at[r, sl][...] = ya.at[r, sl][...] + yb.at[r, sl][...]

        pltpu.sync_copy(ya, out_hbm.at[pl.ds(base + c * CHUNK, CHUNK)])


def _combine(yg, dpack):
    mesh = plsc.VectorSubcoreMesh(core_axis_name="core",
                                  subcore_axis_name="subcore")
    f = pl.kernel(
        _combine_body,
        out_type=jax.ShapeDtypeStruct((T, D), jnp.float32),
        mesh=mesh,
        scratch_types=[
            pltpu.VMEM((1, 2 * TPW), jnp.int32),
            pltpu.VMEM((CHUNK, D), jnp.float32),
            pltpu.VMEM((CHUNK, D), jnp.float32),
        ],
        compiler_params=pltpu.CompilerParams(needs_layout_passes=False),
    )
    return f(yg, dpack)


def kernel(x, Wg, W1, b1, W2, b2):
    dpack, gpack, ef, sw, nxt, slot = _router(x, Wg)
    xin, garr = _dispatch(x, dpack, gpack)
    yg = _ffn(ef.reshape(NTILES), sw.reshape(NTILES), nxt.reshape(NTILES),
              slot.reshape(NTILES), xin, garr.reshape(NROWS, 1),
              W1, b1.reshape(E, 1, H), W2, b2.reshape(E, 1, D))
    return _combine(yg, dpack)


# speedup vs baseline: 1.0876x; 1.0876x over previous
import jax
import jax.numpy as jnp
from jax import lax
from jax.experimental import pallas as pl
from jax.experimental.pallas import tpu as pltpu
from jax.experimental.pallas import tpu_sc as plsc

T = 2048
D = 1024
H = 2048
E = 8
TILE_M = 256
NROWS = T * 2 + E * TILE_M
NTILES = NROWS // TILE_M
CSBLK = 256

NC = 2
NS = 16
NW = NC * NS
TPW = T // NW
CHUNK = 16


def _router_body(x_ref, wg_ref, dp_ref, gp_ref, meta_ref):
    x = x_ref[...]
    wg = wg_ref[...]
    logits = jnp.dot(x, wg, preferred_element_type=jnp.float32)
    m = jnp.max(logits, axis=1, keepdims=True)
    ex = jnp.exp(logits - m)
    probs = ex / jnp.sum(ex, axis=1, keepdims=True)

    eids = lax.broadcasted_iota(jnp.int32, (T, E), 1)
    v0 = jnp.max(probs, axis=1, keepdims=True)
    i0 = jnp.min(jnp.where(probs == v0, eids, E), axis=1, keepdims=True)
    oh0 = (eids == i0).astype(jnp.float32)
    probs1 = jnp.where(eids == i0, -1.0, probs)
    v1 = jnp.max(probs1, axis=1, keepdims=True)
    i1 = jnp.min(jnp.where(probs1 == v1, eids, E), axis=1, keepdims=True)
    oh1 = (eids == i1).astype(jnp.float32)

    s = v0 + v1

    mem = oh0 + oh1
    li = lax.broadcasted_iota(jnp.int32, (CSBLK, CSBLK), 0)
    lj = lax.broadcasted_iota(jnp.int32, (CSBLK, CSBLK), 1)
    ltri = (li >= lj).astype(jnp.float32)
    carry = jnp.zeros((1, E), dtype=jnp.float32)
    blocks = []
    for b in range(T // CSBLK):
        mb = lax.slice(mem, (b * CSBLK, 0), ((b + 1) * CSBLK, E))
        cb = jnp.dot(ltri, mb, preferred_element_type=jnp.float32) + carry
        carry = lax.slice(cb, (CSBLK - 1, 0), (CSBLK, E))
        blocks.append(cb)
    csum = jnp.concatenate(blocks, axis=0)

    counts = carry
    padded = (jnp.floor((counts + (TILE_M - 1)) * (1.0 / TILE_M))) * TILE_M
    ei = lax.broadcasted_iota(jnp.int32, (E, E), 0)
    ej = lax.broadcasted_iota(jnp.int32, (E, E), 1)
    utri = (ei <= ej).astype(jnp.float32)
    ends = jnp.dot(padded, utri, preferred_element_type=jnp.float32)
    offsets = ends - padded

    off0 = jnp.sum(offsets * oh0, axis=1, keepdims=True)
    off1 = jnp.sum(offsets * oh1, axis=1, keepdims=True)
    c0 = jnp.sum(csum * oh0, axis=1, keepdims=True)
    c1 = jnp.sum(csum * oh1, axis=1, keepdims=True)
    d0 = (off0 + c0).astype(jnp.int32) - 1
    d1 = (off1 + c1).astype(jnp.int32) - 1
    dp_ref[...] = jnp.concatenate(
        [d0.reshape(NW, TPW), d1.reshape(NW, TPW)], axis=1)
    gp_ref[...] = jnp.concatenate(
        [(v0 / s).reshape(NW, TPW), (v1 / s).reshape(NW, TPW)], axis=1)

    tstart = (lax.broadcasted_iota(jnp.int32, (NTILES, E), 0)
              * TILE_M).astype(jnp.float32)
    ef = jnp.minimum(
        jnp.sum((tstart >= ends).astype(jnp.int32), axis=1, keepdims=True),
        E - 1)

    efprev = jnp.concatenate(
        [jnp.full((1, 1), -1, jnp.int32),
         lax.slice(ef, (0, 0), (NTILES - 1, 1))], axis=0)
    sw = (ef != efprev).astype(jnp.int32)

    ohT = (lax.broadcasted_iota(jnp.int32, (NTILES, E), 1)
           == ef).astype(jnp.float32)
    present = jnp.where(
        (padded > 0.5)
        | (lax.broadcasted_iota(jnp.int32, (1, E), 1) == E - 1), 1.0, 0.0)
    exc = jnp.dot(present, (ei < ej).astype(jnp.float32),
                  preferred_element_type=jnp.float32)
    rid = jnp.sum(exc * ohT, axis=1, keepdims=True).astype(jnp.int32)
    slot = lax.rem(rid, 2)

    eye = (ei == ej).astype(jnp.float32)
    present_col = lax.dot_general(eye, present,
                                  (((1,), (1,)), ((), ())))
    valt = jnp.where((ei > ej) & (present_col > 0.5),
                     ei.astype(jnp.float32), float(E))
    nxt_of = jnp.min(valt, axis=0, keepdims=True)
    nxt_of = jnp.where(nxt_of > E - 0.5, -1.0, nxt_of)
    nxt = jnp.sum(nxt_of * ohT, axis=1, keepdims=True).astype(jnp.int32)

    used = (lax.slice(tstart, (0, E - 1), (NTILES, E))
            < lax.slice(ends, (0, E - 1), (1, E))).astype(jnp.int32)
    meta_ref[...] = jnp.concatenate([ef, sw, nxt, slot, used], axis=1)


def _router(x, wg):
    return pl.pallas_call(
        _router_body,
        out_shape=[
            jax.ShapeDtypeStruct((NW, 2 * TPW), jnp.int32),
            jax.ShapeDtypeStruct((NW, 2 * TPW), jnp.float32),
            jax.ShapeDtypeStruct((NTILES, 5), jnp.int32),
        ],
        compiler_params=pltpu.CompilerParams(
            vmem_limit_bytes=60 * 1024 * 1024),
    )(x, wg)


def _dispatch_body(x_hbm, dp_hbm, gp_hbm, xin_hbm, garr_hbm,
                   xv, ipk, dpv, gpv, garr_v):
    cid = lax.axis_index("core")
    sid = lax.axis_index("subcore")
    wid = sid * NC + cid
    base = pl.multiple_of(wid * TPW, TPW)

    pltpu.sync_copy(x_hbm.at[pl.ds(base, TPW)], xv)
    pltpu.sync_copy(dp_hbm.at[pl.ds(wid, 1)], ipk)
    for j in range(TPW // 16):
        rows = xv.at[pl.ds(16 * j, 16)]
        pltpu.sync_copy(rows, xin_hbm.at[ipk[0, pl.ds(16 * j, 16)]])
        pltpu.sync_copy(rows, xin_hbm.at[ipk[0, pl.ds(TPW + 16 * j, 16)]])

    @pl.when(wid == 0)
    def _():
        pltpu.sync_copy(dp_hbm, dpv)
        pltpu.sync_copy(gp_hbm, gpv)
        for w in range(NW):
            for j in range(2 * TPW // 16):
                sl = pl.ds(16 * j, 16)
                plsc.store_scatter(garr_v, [dpv[w, sl]], gpv[w, sl])
        pltpu.sync_copy(garr_v, garr_hbm)


def _dispatch(x, dpack, gpack):
    mesh = plsc.VectorSubcoreMesh(core_axis_name="core",
                                  subcore_axis_name="subcore")
    f = pl.kernel(
        _dispatch_body,
        out_type=[
            jax.ShapeDtypeStruct((NROWS, D), jnp.float32),
            jax.ShapeDtypeStruct((NROWS,), jnp.float32),
        ],
        mesh=mesh,
        scratch_types=[
            pltpu.VMEM((TPW, D), jnp.float32),
            pltpu.VMEM((1, 2 * TPW), jnp.int32),
            pltpu.VMEM((NW, 2 * TPW), jnp.int32),
            pltpu.VMEM((NW, 2 * TPW), jnp.float32),
            pltpu.VMEM((NROWS,), jnp.float32),
        ],
        compiler_params=pltpu.CompilerParams(needs_layout_passes=False),
    )
    return f(x, dpack, gpack)


def _ffn_body(meta_ref, x_ref, g_ref, w1_hbm, b1_ref, w2_hbm, b2_ref, o_ref,
              w1v, w2v, sem1, sem2):
    i = pl.program_id(0)
    e = meta_ref[i, 0]
    sl = meta_ref[i, 3]

    @pl.when(i == 0)
    def _():
        pltpu.make_async_copy(w1_hbm.at[e], w1v.at[sl], sem1.at[sl]).start()
        pltpu.make_async_copy(w2_hbm.at[e], w2v.at[sl], sem2.at[sl]).start()

    @pl.when(meta_ref[i, 1] == 1)
    def _():
        pltpu.make_async_copy(w1_hbm.at[e], w1v.at[sl], sem1.at[sl]).wait()
        pltpu.make_async_copy(w2_hbm.at[e], w2v.at[sl], sem2.at[sl]).wait()
        ne = meta_ref[i, 2]

        @pl.when(ne >= 0)
        def _():
            osl = 1 - sl
            pltpu.make_async_copy(w1_hbm.at[ne], w1v.at[osl],
                                  sem1.at[osl]).start()
            pltpu.make_async_copy(w2_hbm.at[ne], w2v.at[osl],
                                  sem2.at[osl]).start()

    @pl.when(meta_ref[i, 4] == 1)
    def _():
        h = jnp.dot(x_ref[...], w1v[sl], preferred_element_type=jnp.float32)
        h = jax.nn.gelu(h + b1_ref[e])
        y = jnp.dot(h, w2v[sl], preferred_element_type=jnp.float32)
        o_ref[...] = (y + b2_ref[e]) * g_ref[...]


def _ffn(meta, xin, garr, w1, b1, w2, b2):
    grid_spec = pltpu.PrefetchScalarGridSpec(
        num_scalar_prefetch=1,
        grid=(NTILES,),
        in_specs=[
            pl.BlockSpec((TILE_M, D), lambda i, *_: (i, 0)),
            pl.BlockSpec((TILE_M, 1), lambda i, *_: (i, 0)),
            pl.BlockSpec(memory_space=pl.ANY),
            pl.BlockSpec((E, 1, H), lambda i, *_: (0, 0, 0)),
            pl.BlockSpec(memory_space=pl.ANY),
            pl.BlockSpec((E, 1, D), lambda i, *_: (0, 0, 0)),
        ],
        out_specs=pl.BlockSpec((TILE_M, D), lambda i, *_: (i, 0)),
        scratch_shapes=[
            pltpu.VMEM((2, D, H), jnp.float32),
            pltpu.VMEM((2, H, D), jnp.float32),
            pltpu.SemaphoreType.DMA((2,)),
            pltpu.SemaphoreType.DMA((2,)),
        ],
    )
    return pl.pallas_call(
        _ffn_body,
        grid_spec=grid_spec,
        out_shape=jax.ShapeDtypeStruct((NROWS, D), jnp.float32),
        compiler_params=pltpu.CompilerParams(
            dimension_semantics=("arbitrary",),
            vmem_limit_bytes=60 * 1024 * 1024),
    )(meta, xin, garr, w1, b1, w2, b2)


def _combine_body(yg_hbm, dp_hbm, out_hbm, ipk, ya0, ya1, yb0, yb1,
                  sa0, sa1, sb0, sb1):
    cid = lax.axis_index("core")
    sid = lax.axis_index("subcore")
    wid = sid * NC + cid
    base = pl.multiple_of(wid * TPW, TPW)
    nch = TPW // CHUNK
    yab = (ya0, ya1)
    ybb = (yb0, yb1)
    sab = (sa0, sa1)
    sbb = (sb0, sb1)

    pltpu.sync_copy(dp_hbm.at[pl.ds(wid, 1)], ipk)

    def gathers(c, b):
        ia = ipk.at[0, pl.ds(c * CHUNK, CHUNK)]
        ib = ipk.at[0, pl.ds(TPW + c * CHUNK, CHUNK)]
        ca = pltpu.make_async_copy(yg_hbm.at[ia], yab[b], sab[b])
        cb = pltpu.make_async_copy(yg_hbm.at[ib], ybb[b], sbb[b])
        return ca, cb

    ca, cb = gathers(0, 0)
    ca.start()
    cb.start()
    for c in range(nch):
        b = c % 2
        if c + 1 < nch:
            na, nb = gathers(c + 1, 1 - b)
            na.start()
            nb.start()
        ca, cb = gathers(c, b)
        ca.wait()
        cb.wait()
        ya, yb = yab[b], ybb[b]

        @pl.loop(0, CHUNK)
        def _(r):
            for cc in range(0, D, 16):
                sl = pl.ds(cc, 16)
                ya.at[r, sl][...] = ya.at[r, sl][...] + yb.at[r, sl][...]

        pltpu.sync_copy(ya, out_hbm.at[pl.ds(base + c * CHUNK, CHUNK)])


def _combine(yg, dpack):
    mesh = plsc.VectorSubcoreMesh(core_axis_name="core",
                                  subcore_axis_name="subcore")
    f = pl.kernel(
        _combine_body,
        out_type=jax.ShapeDtypeStruct((T, D), jnp.float32),
        mesh=mesh,
        scratch_types=[
            pltpu.VMEM((1, 2 * TPW), jnp.int32),
            pltpu.VMEM((CHUNK, D), jnp.float32),
            pltpu.VMEM((CHUNK, D), jnp.float32),
            pltpu.VMEM((CHUNK, D), jnp.float32),
            pltpu.VMEM((CHUNK, D), jnp.float32),
            pltpu.SemaphoreType.DMA,
            pltpu.SemaphoreType.DMA,
            pltpu.SemaphoreType.DMA,
            pltpu.SemaphoreType.DMA,
        ],
        compiler_params=pltpu.CompilerParams(needs_layout_passes=False),
    )
    return f(yg, dpack)


def kernel(x, Wg, W1, b1, W2, b2):
    dpack, gpack, meta = _router(x, Wg)
    xin, garr = _dispatch(x, dpack, gpack)
    yg = _ffn(meta, xin, garr.reshape(NROWS, 1),
              W1, b1.reshape(E, 1, H), W2, b2.reshape(E, 1, D))
    return _combine(yg, dpack)


# speedup vs baseline: 1.0886x; 1.0009x over previous
import jax
import jax.numpy as jnp
from jax import lax
from jax.experimental import pallas as pl
from jax.experimental.pallas import tpu as pltpu
from jax.experimental.pallas import tpu_sc as plsc

T = 2048
D = 1024
H = 2048
E = 8
TILE_M = 256
NROWS = T * 2 + E * TILE_M
NTILES = NROWS // TILE_M
CSBLK = 256

NC = 2
NS = 16
NW = NC * NS
TPW = T // NW
CHUNK = 16


def _router_body(x_ref, wg_ref, dp_ref, gp_ref, meta_ref):
    x = x_ref[...]
    wg = wg_ref[...]
    logits = jnp.dot(x, wg, preferred_element_type=jnp.float32)
    m = jnp.max(logits, axis=1, keepdims=True)
    ex = jnp.exp(logits - m)
    probs = ex / jnp.sum(ex, axis=1, keepdims=True)

    eids = lax.broadcasted_iota(jnp.int32, (T, E), 1)
    v0 = jnp.max(probs, axis=1, keepdims=True)
    i0 = jnp.min(jnp.where(probs == v0, eids, E), axis=1, keepdims=True)
    oh0 = (eids == i0).astype(jnp.float32)
    probs1 = jnp.where(eids == i0, -1.0, probs)
    v1 = jnp.max(probs1, axis=1, keepdims=True)
    i1 = jnp.min(jnp.where(probs1 == v1, eids, E), axis=1, keepdims=True)
    oh1 = (eids == i1).astype(jnp.float32)

    s = v0 + v1

    mem = oh0 + oh1
    li = lax.broadcasted_iota(jnp.int32, (CSBLK, CSBLK), 0)
    lj = lax.broadcasted_iota(jnp.int32, (CSBLK, CSBLK), 1)
    ltri = (li >= lj).astype(jnp.float32)
    carry = jnp.zeros((1, E), dtype=jnp.float32)
    blocks = []
    for b in range(T // CSBLK):
        mb = lax.slice(mem, (b * CSBLK, 0), ((b + 1) * CSBLK, E))
        cb = jnp.dot(ltri, mb, preferred_element_type=jnp.float32) + carry
        carry = lax.slice(cb, (CSBLK - 1, 0), (CSBLK, E))
        blocks.append(cb)
    csum = jnp.concatenate(blocks, axis=0)

    counts = carry
    padded = (jnp.floor((counts + (TILE_M - 1)) * (1.0 / TILE_M))) * TILE_M
    ei = lax.broadcasted_iota(jnp.int32, (E, E), 0)
    ej = lax.broadcasted_iota(jnp.int32, (E, E), 1)
    utri = (ei <= ej).astype(jnp.float32)
    ends = jnp.dot(padded, utri, preferred_element_type=jnp.float32)
    offsets = ends - padded

    off0 = jnp.sum(offsets * oh0, axis=1, keepdims=True)
    off1 = jnp.sum(offsets * oh1, axis=1, keepdims=True)
    c0 = jnp.sum(csum * oh0, axis=1, keepdims=True)
    c1 = jnp.sum(csum * oh1, axis=1, keepdims=True)
    d0 = (off0 + c0).astype(jnp.int32) - 1
    d1 = (off1 + c1).astype(jnp.int32) - 1
    dp_ref[...] = jnp.concatenate(
        [d0.reshape(NW, TPW), d1.reshape(NW, TPW)], axis=1)
    gp_ref[...] = jnp.concatenate(
        [(v0 / s).reshape(NW, TPW), (v1 / s).reshape(NW, TPW)], axis=1)

    tstart = (lax.broadcasted_iota(jnp.int32, (NTILES, E), 0)
              * TILE_M).astype(jnp.float32)
    ef = jnp.minimum(
        jnp.sum((tstart >= ends).astype(jnp.int32), axis=1, keepdims=True),
        E - 1)

    efprev = jnp.concatenate(
        [jnp.full((1, 1), -1, jnp.int32),
         lax.slice(ef, (0, 0), (NTILES - 1, 1))], axis=0)
    sw = (ef != efprev).astype(jnp.int32)

    ohT = (lax.broadcasted_iota(jnp.int32, (NTILES, E), 1)
           == ef).astype(jnp.float32)
    present = jnp.where(
        (padded > 0.5)
        | (lax.broadcasted_iota(jnp.int32, (1, E), 1) == E - 1), 1.0, 0.0)
    exc = jnp.dot(present, (ei < ej).astype(jnp.float32),
                  preferred_element_type=jnp.float32)
    rid = jnp.sum(exc * ohT, axis=1, keepdims=True).astype(jnp.int32)
    slot = lax.rem(rid, 2)

    eye = (ei == ej).astype(jnp.float32)
    present_col = lax.dot_general(eye, present,
                                  (((1,), (1,)), ((), ())))
    valt = jnp.where((ei > ej) & (present_col > 0.5),
                     ei.astype(jnp.float32), float(E))
    nxt_of = jnp.min(valt, axis=0, keepdims=True)
    nxt_of = jnp.where(nxt_of > E - 0.5, -1.0, nxt_of)
    nxt = jnp.sum(nxt_of * ohT, axis=1, keepdims=True).astype(jnp.int32)

    used = (lax.slice(tstart, (0, E - 1), (NTILES, E))
            < lax.slice(ends, (0, E - 1), (1, E))).astype(jnp.int32)
    meta_ref[...] = jnp.concatenate([ef, sw, nxt, slot, used], axis=1)


def _router(x, wg):
    return pl.pallas_call(
        _router_body,
        out_shape=[
            jax.ShapeDtypeStruct((NW, 2 * TPW), jnp.int32),
            jax.ShapeDtypeStruct((NW, 2 * TPW), jnp.float32),
            jax.ShapeDtypeStruct((NTILES, 5), jnp.int32),
        ],
        compiler_params=pltpu.CompilerParams(
            vmem_limit_bytes=60 * 1024 * 1024),
    )(x, wg)


def _dispatch_body(x_hbm, dp_hbm, gp_hbm, xin_hbm, garr_hbm,
                   xv, ipk, dpv, gpv, garr_v):
    cid = lax.axis_index("core")
    sid = lax.axis_index("subcore")
    wid = sid * NC + cid
    base = pl.multiple_of(wid * TPW, TPW)

    pltpu.sync_copy(x_hbm.at[pl.ds(base, TPW)], xv)
    pltpu.sync_copy(dp_hbm.at[pl.ds(wid, 1)], ipk)
    for j in range(TPW // 16):
        rows = xv.at[pl.ds(16 * j, 16)]
        pltpu.sync_copy(rows, xin_hbm.at[ipk[0, pl.ds(16 * j, 16)]])
        pltpu.sync_copy(rows, xin_hbm.at[ipk[0, pl.ds(TPW + 16 * j, 16)]])

    @pl.when(wid == 0)
    def _():
        pltpu.sync_copy(dp_hbm, dpv)
        pltpu.sync_copy(gp_hbm, gpv)
        for w in range(NW):
            for j in range(2 * TPW // 16):
                sl = pl.ds(16 * j, 16)
                plsc.store_scatter(garr_v, [dpv[w, sl]], gpv[w, sl])
        pltpu.sync_copy(garr_v, garr_hbm)


def _dispatch(x, dpack, gpack):
    mesh = plsc.VectorSubcoreMesh(core_axis_name="core",
                                  subcore_axis_name="subcore")
    f = pl.kernel(
        _dispatch_body,
        out_type=[
            jax.ShapeDtypeStruct((NROWS, D), jnp.float32),
            jax.ShapeDtypeStruct((NROWS,), jnp.float32),
        ],
        mesh=mesh,
        scratch_types=[
            pltpu.VMEM((TPW, D), jnp.float32),
            pltpu.VMEM((1, 2 * TPW), jnp.int32),
            pltpu.VMEM((NW, 2 * TPW), jnp.int32),
            pltpu.VMEM((NW, 2 * TPW), jnp.float32),
            pltpu.VMEM((NROWS,), jnp.float32),
        ],
        compiler_params=pltpu.CompilerParams(needs_layout_passes=False),
    )
    return f(x, dpack, gpack)


def _ffn_body(meta_ref, x_ref, g_ref, w1_hbm, b1_ref, w2_hbm, b2_ref, o_ref,
              w1v0, w1v1, w2v0, w2v1, sem1, sem2):
    i = pl.program_id(0)
    e = meta_ref[i, 0]
    sl = meta_ref[i, 3]
    w1b = (w1v0, w1v1)
    w2b = (w2v0, w2v1)

    def start_into(ne, s):
        pltpu.make_async_copy(w1_hbm.at[ne], w1b[s], sem1.at[s]).start()
        pltpu.make_async_copy(w2_hbm.at[ne], w2b[s], sem2.at[s]).start()

    def wait_into(ne, s):
        pltpu.make_async_copy(w1_hbm.at[ne], w1b[s], sem1.at[s]).wait()
        pltpu.make_async_copy(w2_hbm.at[ne], w2b[s], sem2.at[s]).wait()

    @pl.when(i == 0)
    def _():
        start_into(e, 0)

    @pl.when(meta_ref[i, 1] == 1)
    def _():
        ne = meta_ref[i, 2]

        @pl.when(sl == 0)
        def _():
            wait_into(e, 0)

            @pl.when(ne >= 0)
            def _():
                start_into(ne, 1)

        @pl.when(sl == 1)
        def _():
            wait_into(e, 1)

            @pl.when(ne >= 0)
            def _():
                start_into(ne, 0)

    def compute(w1v, w2v):
        h = jnp.dot(x_ref[...], w1v[...], preferred_element_type=jnp.float32)
        h = jax.nn.gelu(h + b1_ref[e])
        y = jnp.dot(h, w2v[...], preferred_element_type=jnp.float32)
        o_ref[...] = (y + b2_ref[e]) * g_ref[...]

    @pl.when((meta_ref[i, 4] == 1) & (sl == 0))
    def _():
        compute(w1v0, w2v0)

    @pl.when((meta_ref[i, 4] == 1) & (sl == 1))
    def _():
        compute(w1v1, w2v1)


def _ffn(meta, xin, garr, w1, b1, w2, b2):
    grid_spec = pltpu.PrefetchScalarGridSpec(
        num_scalar_prefetch=1,
        grid=(NTILES,),
        in_specs=[
            pl.BlockSpec((TILE_M, D), lambda i, *_: (i, 0)),
            pl.BlockSpec((TILE_M, 1), lambda i, *_: (i, 0)),
            pl.BlockSpec(memory_space=pl.ANY),
            pl.BlockSpec((E, 1, H), lambda i, *_: (0, 0, 0)),
            pl.BlockSpec(memory_space=pl.ANY),
            pl.BlockSpec((E, 1, D), lambda i, *_: (0, 0, 0)),
        ],
        out_specs=pl.BlockSpec((TILE_M, D), lambda i, *_: (i, 0)),
        scratch_shapes=[
            pltpu.VMEM((D, H), jnp.float32),
            pltpu.VMEM((D, H), jnp.float32),
            pltpu.VMEM((H, D), jnp.float32),
            pltpu.VMEM((H, D), jnp.float32),
            pltpu.SemaphoreType.DMA((2,)),
            pltpu.SemaphoreType.DMA((2,)),
        ],
    )
    return pl.pallas_call(
        _ffn_body,
        grid_spec=grid_spec,
        out_shape=jax.ShapeDtypeStruct((NROWS, D), jnp.float32),
        compiler_params=pltpu.CompilerParams(
            dimension_semantics=("arbitrary",),
            vmem_limit_bytes=60 * 1024 * 1024),
    )(meta, xin, garr, w1, b1, w2, b2)


def _combine_body(yg_hbm, dp_hbm, out_hbm, ipk, ya0, ya1, yb0, yb1,
                  sa0, sa1, sb0, sb1):
    cid = lax.axis_index("core")
    sid = lax.axis_index("subcore")
    wid = sid * NC + cid
    base = pl.multiple_of(wid * TPW, TPW)
    nch = TPW // CHUNK
    yab = (ya0, ya1)
    ybb = (yb0, yb1)
    sab = (sa0, sa1)
    sbb = (sb0, sb1)

    pltpu.sync_copy(dp_hbm.at[pl.ds(wid, 1)], ipk)

    def gathers(c, b):
        ia = ipk.at[0, pl.ds(c * CHUNK, CHUNK)]
        ib = ipk.at[0, pl.ds(TPW + c * CHUNK, CHUNK)]
        ca = pltpu.make_async_copy(yg_hbm.at[ia], yab[b], sab[b])
        cb = pltpu.make_async_copy(yg_hbm.at[ib], ybb[b], sbb[b])
        return ca, cb

    ca, cb = gathers(0, 0)
    ca.start()
    cb.start()
    for c in range(nch):
        b = c % 2
        if c + 1 < nch:
            na, nb = gathers(c + 1, 1 - b)
            na.start()
            nb.start()
        ca, cb = gathers(c, b)
        ca.wait()
        cb.wait()
        ya, yb = yab[b], ybb[b]

        @pl.loop(0, CHUNK)
        def _(r):
            for cc in range(0, D, 16):
                sl = pl.ds(cc, 16)
                ya.at[r, sl][...] = ya.at[r, sl][...] + yb.at[r, sl][...]

        pltpu.sync_copy(ya, out_hbm.at[pl.ds(base + c * CHUNK, CHUNK)])


def _combine(yg, dpack):
    mesh = plsc.VectorSubcoreMesh(core_axis_name="core",
                                  subcore_axis_name="subcore")
    f = pl.kernel(
        _combine_body,
        out_type=jax.ShapeDtypeStruct((T, D), jnp.float32),
        mesh=mesh,
        scratch_types=[
            pltpu.VMEM((1, 2 * TPW), jnp.int32),
            pltpu.VMEM((CHUNK, D), jnp.float32),
            pltpu.VMEM((CHUNK, D), jnp.float32),
            pltpu.VMEM((CHUNK, D), jnp.float32),
            pltpu.VMEM((CHUNK, D), jnp.float32),
            pltpu.SemaphoreType.DMA,
            pltpu.SemaphoreType.DMA,
            pltpu.SemaphoreType.DMA,
            pltpu.SemaphoreType.DMA,
        ],
        compiler_params=pltpu.CompilerParams(needs_layout_passes=False),
    )
    return f(yg, dpack)


def kernel(x, Wg, W1, b1, W2, b2):
    dpack, gpack, meta = _router(x, Wg)
    xin, garr = _dispatch(x, dpack, gpack)
    yg = _ffn(meta, xin, garr.reshape(NROWS, 1),
              W1, b1.reshape(E, 1, H), W2, b2.reshape(E, 1, D))
    return _combine(yg, dpack)
